# Initial kernel scaffold; baseline (speedup 1.0000x reference)
import jax
import jax.numpy as jnp
from jax.experimental import pallas as pl


def kernel(x, seqnumkmin, Wc, bc):
    raise NotImplementedError("write your pallas kernel here")


# speedup vs baseline: 13.9627x; 13.9627x over previous
import functools

import jax
import jax.numpy as jnp
from jax import lax
from jax.experimental import pallas as pl
from jax.experimental.pallas import tpu as pltpu
from jax.experimental.pallas import tpu_sc as plsc

B = 4
Q = 10
C = 2
H = 128
W = 128
K = 9
MAXJ = 720
NLAT = 361
N = NLAT * MAXJ
NODES = Q * H * W
BC = B * C

NC = 2
NS = 16
NW = NC * NS
PW = 8192
NPAD = NW * PW
SP = 128
NSTEP = PW // SP
SEG = SP * BC // 16


def _sc_combine(table, idx_t, wb):
    mesh = plsc.VectorSubcoreMesh(
        core_axis_name="c", subcore_axis_name="s",
        num_cores=NC, num_subcores=NS)

    @functools.partial(
        pl.kernel,
        out_type=jax.ShapeDtypeStruct((NPAD * BC,), jnp.float32),
        mesh=mesh,
        compiler_params=pltpu.CompilerParams(
            needs_layout_passes=False, use_tc_tiling_on_sc=False),
        scratch_types=[
            pltpu.VMEM((K, SP), jnp.int32),
            [pltpu.VMEM((SP, BC), jnp.float32)] * K,
            pltpu.VMEM((SP * BC,), jnp.float32),
            pltpu.VMEM((K + 1, 16), jnp.float32),
            pltpu.SemaphoreType.DMA,
        ],
    )
    def body(tab_hbm, idx_hbm, wb_hbm, out_hbm, idx_v, bufs, accbuf, wb_v, sem):
        wid = lax.axis_index("s") * NC + lax.axis_index("c")
        pltpu.sync_copy(wb_hbm, wb_v)
        ws = [wb_v[k] for k in range(K)]
        bias_v = wb_v[K]
        iota = lax.iota(jnp.int32, 16)
        rowsel = jnp.right_shift(iota, 3)
        colsel = jnp.bitwise_and(iota, 7)

        def step(i, _):
            off = wid * PW + i * SP
            pltpu.sync_copy(idx_hbm.at[:, pl.ds(off, SP)], idx_v)
            descs = [
                pltpu.async_copy(tab_hbm.at[idx_v.at[k]], bufs[k], sem)
                for k in range(K)
            ]
            for d in descs:
                d.wait()

            def seg(s, _):
                row = 2 * s + rowsel
                acc = bias_v
                for k in range(K):
                    v = plsc.load_gather(bufs[k], [row, colsel])
                    acc = acc + v * ws[k]
                accbuf[pl.ds(s * 16, 16)] = acc
                return 0

            lax.fori_loop(0, SEG, seg, 0, unroll=2)
            pltpu.sync_copy(accbuf, out_hbm.at[pl.ds(off * BC, SP * BC)])
            return 0

        lax.fori_loop(0, NSTEP, step, 0)

    return body(table, idx_t, wb)


def kernel(x, seqnumkmin, Wc, bc):
    table = x.reshape(B, Q, C, H, W).transpose(1, 3, 4, 0, 2).reshape(NODES, BC)
    idx_t = jnp.pad(seqnumkmin, ((0, NPAD - N), (0, 0))).T
    w = Wc.reshape(C, K)
    wb = jnp.concatenate([
        jnp.tile(w.T, (1, 16 // C)),
        jnp.tile(bc, 16 // C)[None],
    ], axis=0)
    out8 = _sc_combine(table, idx_t, wb)
    out = out8[: N * BC].reshape(N, B, C).transpose(1, 2, 0)
    return out.reshape(B, C, NLAT, MAXJ)


# speedup vs baseline: 36.0580x; 2.5825x over previous
import functools

import jax
import jax.numpy as jnp
from jax import lax
from jax.experimental import pallas as pl
from jax.experimental.pallas import tpu as pltpu
from jax.experimental.pallas import tpu_sc as plsc

B = 4
Q = 10
C = 2
H = 128
W = 128
K = 9
MAXJ = 720
NLAT = 361
N = NLAT * MAXJ
NODES = Q * H * W
BC = B * C

NC = 2
NS = 16
NW = NC * NS
PW = 8192
SP = 128
NSTEP = PW // SP
NT = NSTEP // 2


def _sc_combine(table, idx_raw, wbs):
    mesh = plsc.VectorSubcoreMesh(
        core_axis_name="c", subcore_axis_name="s",
        num_cores=NC, num_subcores=NS)

    @functools.partial(
        pl.kernel,
        out_type=jax.ShapeDtypeStruct((BC, N), jnp.float32),
        mesh=mesh,
        compiler_params=pltpu.CompilerParams(
            needs_layout_passes=False, use_tc_tiling_on_sc=False),
        scratch_types=[
            [pltpu.VMEM((SP, K), jnp.int32)] * 2,
            [[pltpu.VMEM((SP,), jnp.int32)] * K] * 2,
            [[pltpu.VMEM((SP, BC), jnp.float32)] * K] * 2,
            [pltpu.VMEM((BC, SP), jnp.float32)] * 2,
            pltpu.VMEM((K + 1, BC, 16), jnp.float32),
            [pltpu.SemaphoreType.DMA] * 2,
            [pltpu.SemaphoreType.DMA] * 2,
            [pltpu.SemaphoreType.DMA] * 2,
        ],
    )
    def body(tab_hbm, idx_hbm, wbs_hbm, out_hbm,
             idx_raw_v, idx_vs, bufs, accT, wsm, isem, gsem, osem):
        wid = lax.axis_index("s") * NC + lax.axis_index("c")
        pltpu.sync_copy(wbs_hbm, wsm)
        iota = lax.iota(jnp.int32, 16)
        ksel = [jnp.full((16,), k, jnp.int32) for k in range(K)]
        jsel = [jnp.full((16,), j, jnp.int32) for j in range(BC)]

        def off_of(i):
            return jnp.minimum(wid * PW + i * SP, N - SP)

        def prefetch_idx(i, p):
            pltpu.async_copy(
                idx_hbm.at[pl.ds(off_of(i), SP), :], idx_raw_v[p], isem[p])

        def launch(i, p):
            pltpu.make_async_copy(
                idx_hbm.at[pl.ds(0, SP), :], idx_raw_v[p], isem[p]).wait()

            def tr(s8, _):
                pts = s8 * 16 + iota
                for k in range(K):
                    v = plsc.load_gather(idx_raw_v[p], [pts, ksel[k]])
                    idx_vs[p][k][pl.ds(s8 * 16, 16)] = v
                return 0

            lax.fori_loop(0, SP // 16, tr, 0)
            for k in range(K):
                pltpu.async_copy(
                    tab_hbm.at[idx_vs[p][k]], bufs[p][k], gsem[p])

        def drain_out(p):
            pltpu.make_async_copy(
                out_hbm.at[:, pl.ds(0, SP)], accT[p], osem[p]).wait()

        def finish(i, p):
            for k in range(K):
                pltpu.make_async_copy(
                    tab_hbm.at[idx_vs[p][k]], bufs[p][k], gsem[p]).wait()

            @pl.when(i >= 2)
            def _():
                drain_out(p)

            for j in range(BC):
                bias_vj = wsm[K, j]
                wvs = [wsm[k, j] for k in range(K)]

                def seg(s8, _):
                    pts = s8 * 16 + iota
                    acc = bias_vj
                    for k in range(K):
                        v = plsc.load_gather(bufs[p][k], [pts, jsel[j]])
                        acc = acc + v * wvs[k]
                    accT[p][j, pl.ds(s8 * 16, 16)] = acc
                    return 0

                lax.fori_loop(0, SP // 16, seg, 0)
            off = off_of(i)
            for j in range(BC):
                pltpu.async_copy(
                    accT[p].at[j], out_hbm.at[j, pl.ds(off, SP)], osem[p])

        prefetch_idx(0, 0)
        launch(0, 0)
        prefetch_idx(1, 1)

        def outer(t, _):
            prefetch_idx(2 * t + 2, 0)
            launch(2 * t + 1, 1)
            finish(2 * t, 0)
            launch(2 * t + 2, 0)
            prefetch_idx(2 * t + 3, 1)
            finish(2 * t + 1, 1)
            return 0

        lax.fori_loop(0, NT - 1, outer, 0)
        launch(NSTEP - 1, 1)
        finish(NSTEP - 2, 0)
        finish(NSTEP - 1, 1)
        drain_out(0)
        drain_out(1)

    return body(table, idx_raw, wbs)


def kernel(x, seqnumkmin, Wc, bc):
    table = x.reshape(B, Q, C, H, W).transpose(1, 3, 4, 0, 2).reshape(NODES, BC)
    w = Wc.reshape(C, K)
    jc = jnp.arange(BC) % C
    wbs = jnp.concatenate([
        w.T[:, jc],
        bc[jc][None],
    ])[:, :, None] * jnp.ones((1, 1, 16), jnp.float32)
    out = _sc_combine(table, seqnumkmin, wbs)
    return out.reshape(B, C, NLAT, MAXJ)


# speedup vs baseline: 41.3301x; 1.1462x over previous
import functools

import jax
import jax.numpy as jnp
from jax import lax
from jax.experimental import pallas as pl
from jax.experimental.pallas import tpu as pltpu
from jax.experimental.pallas import tpu_sc as plsc

B = 4
Q = 10
C = 2
H = 128
W = 128
K = 9
MAXJ = 720
NLAT = 361
N = NLAT * MAXJ
NODES = Q * H * W
BC = B * C

NC = 2
NS = 16
NW = NC * NS
PW = 8192
SP = 128
NSTEP = PW // SP
NT = NSTEP // 2


def _sc_combine(table, idx_raw, wbs):
    mesh = plsc.VectorSubcoreMesh(
        core_axis_name="c", subcore_axis_name="s",
        num_cores=NC, num_subcores=NS)

    @functools.partial(
        pl.kernel,
        out_type=jax.ShapeDtypeStruct((BC, N), jnp.float32),
        mesh=mesh,
        compiler_params=pltpu.CompilerParams(
            needs_layout_passes=False, use_tc_tiling_on_sc=False),
        scratch_types=[
            [pltpu.VMEM((SP * K,), jnp.int32)] * 2,
            [[pltpu.VMEM((SP,), jnp.int32)] * K] * 2,
            [[pltpu.VMEM((SP, BC), jnp.float32)] * K] * 2,
            [pltpu.VMEM((BC, SP), jnp.float32)] * 2,
            pltpu.VMEM((K + 1, BC, 16), jnp.float32),
            [pltpu.SemaphoreType.DMA] * 2,
            [pltpu.SemaphoreType.DMA] * 2,
            [pltpu.SemaphoreType.DMA] * 2,
        ],
    )
    def body(tab_hbm, idx_hbm, wbs_hbm, out_hbm,
             idx_raw_v, idx_vs, bufs, accT, wsm, isem, gsem, osem):
        wid = lax.axis_index("s") * NC + lax.axis_index("c")
        pltpu.sync_copy(wbs_hbm, wsm)
        iota = lax.iota(jnp.int32, 16)
        jsel = [jnp.full((16,), j, jnp.int32) for j in range(BC)]

        def off_of(i):
            return jnp.minimum(wid * PW + i * SP, N - SP)

        iota9 = iota * K

        def prefetch_idx(i, p):
            pltpu.async_copy(
                idx_hbm.at[pl.ds(off_of(i) * K, SP * K)], idx_raw_v[p], isem[p])

        def launch(i, p):
            pltpu.make_async_copy(
                idx_hbm.at[pl.ds(0, SP * K)], idx_raw_v[p], isem[p]).wait()

            def tr(s8, _):
                for k in range(K):
                    v = plsc.load_gather(
                        idx_raw_v[p], [s8 * (16 * K) + k + iota9])
                    idx_vs[p][k][pl.ds(s8 * 16, 16)] = v
                return 0

            lax.fori_loop(0, SP // 16, tr, 0)
            for k in range(K):
                pltpu.async_copy(
                    tab_hbm.at[idx_vs[p][k]], bufs[p][k], gsem[p])

        def drain_out(p):
            pltpu.make_async_copy(
                out_hbm.at[:, pl.ds(0, SP)], accT[p], osem[p]).wait()

        def finish(i, p):
            for k in range(K):
                pltpu.make_async_copy(
                    tab_hbm.at[idx_vs[p][k]], bufs[p][k], gsem[p]).wait()

            @pl.when(i >= 2)
            def _():
                drain_out(p)

            for j in range(BC):
                bias_vj = wsm[K, j]
                wvs = [wsm[k, j] for k in range(K)]

                def seg(s8, _):
                    pts = s8 * 16 + iota
                    acc = bias_vj
                    for k in range(K):
                        v = plsc.load_gather(bufs[p][k], [pts, jsel[j]])
                        acc = acc + v * wvs[k]
                    accT[p][j, pl.ds(s8 * 16, 16)] = acc
                    return 0

                lax.fori_loop(0, SP // 16, seg, 0)
            off = off_of(i)
            for j in range(BC):
                pltpu.async_copy(
                    accT[p].at[j], out_hbm.at[j, pl.ds(off, SP)], osem[p])

        prefetch_idx(0, 0)
        launch(0, 0)
        prefetch_idx(1, 1)

        def outer(t, _):
            prefetch_idx(2 * t + 2, 0)
            launch(2 * t + 1, 1)
            finish(2 * t, 0)
            launch(2 * t + 2, 0)
            prefetch_idx(2 * t + 3, 1)
            finish(2 * t + 1, 1)
            return 0

        lax.fori_loop(0, NT - 1, outer, 0)
        launch(NSTEP - 1, 1)
        finish(NSTEP - 2, 0)
        finish(NSTEP - 1, 1)
        drain_out(0)
        drain_out(1)

    return body(table, idx_raw, wbs)


def kernel(x, seqnumkmin, Wc, bc):
    table = x.reshape(B, Q, C, H, W).transpose(1, 3, 4, 0, 2).reshape(NODES, BC)
    w = Wc.reshape(C, K)
    jc = jnp.arange(BC) % C
    wbs = jnp.concatenate([
        w.T[:, jc],
        bc[jc][None],
    ])[:, :, None] * jnp.ones((1, 1, 16), jnp.float32)
    out = _sc_combine(table, seqnumkmin.reshape(-1), wbs)
    return out.reshape(B, C, NLAT, MAXJ)


# speedup vs baseline: 46.8587x; 1.1338x over previous
import functools

import jax
import jax.numpy as jnp
from jax import lax
from jax.experimental import pallas as pl
from jax.experimental.pallas import tpu as pltpu
from jax.experimental.pallas import tpu_sc as plsc

B = 4
Q = 10
C = 2
H = 128
W = 128
K = 9
MAXJ = 720
NLAT = 361
N = NLAT * MAXJ
NODES = Q * H * W
BC = B * C

NC = 2
NS = 16
NW = NC * NS
PW = 8192
SP = 128
NSTEP = PW // SP
NT = NSTEP // 2
NPW = NODES // NS
NB = 1024


def _sc_run(planes, idx_raw, wbs):
    mesh = plsc.VectorSubcoreMesh(
        core_axis_name="c", subcore_axis_name="s",
        num_cores=NC, num_subcores=NS)

    @functools.partial(
        pl.kernel,
        out_type=jax.ShapeDtypeStruct((BC * N,), jnp.float32),
        mesh=mesh,
        compiler_params=pltpu.CompilerParams(
            needs_layout_passes=False, use_tc_tiling_on_sc=False),
        scratch_types=[
            pltpu.VMEM_SHARED((NODES, BC), jnp.float32),
            [pltpu.VMEM((NB,), jnp.float32)] * BC,
            pltpu.VMEM((NB, BC), jnp.float32),
            [pltpu.VMEM((SP * K,), jnp.int32)] * 2,
            [[pltpu.VMEM((SP,), jnp.int32)] * K] * 2,
            [[pltpu.VMEM((SP, BC), jnp.float32)] * K] * 2,
            [pltpu.VMEM((BC, SP), jnp.float32)] * 2,
            pltpu.VMEM((K + 1, BC, 16), jnp.float32),
            pltpu.SemaphoreType.DMA,
            [pltpu.SemaphoreType.DMA] * 2,
            [pltpu.SemaphoreType.DMA] * 2,
            [pltpu.SemaphoreType.DMA] * 2,
        ],
    )
    def body(p0, p1, p2, p3, p4, p5, p6, p7, idx_hbm, wbs_hbm, out_hbm,
             stab, pin, tblk, idx_raw_v, idx_vs, bufs, accT, wsm,
             bsem, isem, gsem, osem):
        planes_hbm = [p0, p1, p2, p3, p4, p5, p6, p7]
        sid = lax.axis_index("s")
        wid = sid * NC + lax.axis_index("c")
        pltpu.sync_copy(wbs_hbm, wsm)
        iota = lax.iota(jnp.int32, 16)
        jsel = [jnp.full((16,), j, jnp.int32) for j in range(BC)]

        def build_blk(blk, _):
            base = sid * NPW + blk * NB
            for j in range(BC):
                pltpu.async_copy(
                    planes_hbm[j].at[pl.ds(base, NB)], pin[j], bsem)
            for j in range(BC):
                pltpu.make_async_copy(
                    planes_hbm[j].at[pl.ds(0, NB)], pin[j], bsem).wait()

            def ilv(s16, _):
                pts = s16 * 16 + iota
                for j in range(BC):
                    v = pin[j][pl.ds(s16 * 16, 16)]
                    plsc.store_scatter(tblk, [pts, jsel[j]], v)
                return 0

            lax.fori_loop(0, NB // 16, ilv, 0)
            pltpu.sync_copy(tblk, stab.at[pl.ds(base, NB), :])
            return 0

        lax.fori_loop(0, NPW // NB, build_blk, 0)
        plsc.subcore_barrier()

        iota9 = iota * K

        def off_of(i):
            return jnp.minimum(wid * PW + i * SP, N - SP)

        def prefetch_idx(i, p):
            pltpu.async_copy(
                idx_hbm.at[pl.ds(off_of(i) * K, SP * K)], idx_raw_v[p], isem[p])

        def launch(i, p):
            pltpu.make_async_copy(
                idx_hbm.at[pl.ds(0, SP * K)], idx_raw_v[p], isem[p]).wait()

            def tr(s8, _):
                for k in range(K):
                    v = plsc.load_gather(
                        idx_raw_v[p], [s8 * (16 * K) + k + iota9])
                    idx_vs[p][k][pl.ds(s8 * 16, 16)] = v
                return 0

            lax.fori_loop(0, SP // 16, tr, 0)
            for k in range(K):
                pltpu.async_copy(
                    stab.at[idx_vs[p][k]], bufs[p][k], gsem[p])

        def drain_out(p):
            for j in range(BC):
                pltpu.make_async_copy(
                    out_hbm.at[pl.ds(0, SP)], accT[p].at[j], osem[p]).wait()

        def finish(i, p):
            for k in range(K):
                pltpu.make_async_copy(
                    stab.at[idx_vs[p][k]], bufs[p][k], gsem[p]).wait()

            @pl.when(i >= 2)
            def _():
                drain_out(p)

            for j in range(BC):
                bias_vj = wsm[K, j]
                wvs = [wsm[k, j] for k in range(K)]

                def seg(s8, _):
                    pts = s8 * 16 + iota
                    acc = bias_vj
                    for k in range(K):
                        v = plsc.load_gather(bufs[p][k], [pts, jsel[j]])
                        acc = acc + v * wvs[k]
                    accT[p][j, pl.ds(s8 * 16, 16)] = acc
                    return 0

                lax.fori_loop(0, SP // 16, seg, 0)
            off = off_of(i)
            for j in range(BC):
                pltpu.async_copy(
                    accT[p].at[j], out_hbm.at[pl.ds(j * N + off, SP)], osem[p])

        prefetch_idx(0, 0)
        launch(0, 0)
        prefetch_idx(1, 1)

        def outer(t, _):
            prefetch_idx(2 * t + 2, 0)
            launch(2 * t + 1, 1)
            finish(2 * t, 0)
            launch(2 * t + 2, 0)
            prefetch_idx(2 * t + 3, 1)
            finish(2 * t + 1, 1)
            return 0

        lax.fori_loop(0, NT - 1, outer, 0)
        launch(NSTEP - 1, 1)
        finish(NSTEP - 2, 0)
        finish(NSTEP - 1, 1)
        drain_out(0)
        drain_out(1)

    return body(*planes, idx_raw, wbs)


def kernel(x, seqnumkmin, Wc, bc):
    xp = x.reshape(B, Q, C, H * W)
    planes = [xp[b, :, c, :].reshape(-1) for b in range(B) for c in range(C)]
    w = Wc.reshape(C, K)
    jc = jnp.arange(BC) % C
    wbs = jnp.concatenate([
        w.T[:, jc],
        bc[jc][None],
    ])[:, :, None] * jnp.ones((1, 1, 16), jnp.float32)
    out = _sc_run(planes, seqnumkmin.reshape(-1), wbs)
    return out.reshape(B, C, NLAT, MAXJ)


# speedup vs baseline: 67.6430x; 1.4436x over previous
import functools

import jax
import jax.numpy as jnp
from jax import lax
from jax.experimental import pallas as pl
from jax.experimental.pallas import tpu as pltpu
from jax.experimental.pallas import tpu_sc as plsc

B = 4
Q = 10
C = 2
H = 128
W = 128
K = 9
MAXJ = 720
NLAT = 361
N = NLAT * MAXJ
NODES = Q * H * W
BC = B * C

NC = 2
NS = 16
NW = NC * NS
PW = 8192
SP = 128
NSTEP = PW // SP
NT = NSTEP // 2
NPW = NODES // NS
NB = 1024


def _sc_run(planes, idx_raw, wbs):
    mesh = plsc.VectorSubcoreMesh(
        core_axis_name="c", subcore_axis_name="s",
        num_cores=NC, num_subcores=NS)

    @functools.partial(
        pl.kernel,
        out_type=jax.ShapeDtypeStruct((BC * N,), jnp.float32),
        mesh=mesh,
        compiler_params=pltpu.CompilerParams(
            needs_layout_passes=False, use_tc_tiling_on_sc=False),
        scratch_types=[
            pltpu.VMEM_SHARED((NODES, BC), jnp.float32),
            [pltpu.VMEM((NB,), jnp.float32)] * BC,
            pltpu.VMEM((NB, BC), jnp.float32),
            [pltpu.VMEM((K, SP), jnp.int32)] * 2,
            [[pltpu.VMEM((SP, BC), jnp.float32)] * K] * 2,
            [pltpu.VMEM((BC, SP), jnp.float32)] * 2,
            pltpu.VMEM((K + 1, BC, 16), jnp.float32),
            pltpu.SemaphoreType.DMA,
            [pltpu.SemaphoreType.DMA] * 2,
            [pltpu.SemaphoreType.DMA] * 2,
            [pltpu.SemaphoreType.DMA] * 2,
        ],
    )
    def body(p0, p1, p2, p3, p4, p5, p6, p7, idx_hbm, wbs_hbm, out_hbm,
             stab, pin, tblk, idx_v, bufs, accT, wsm,
             bsem, isem, gsem, osem):
        planes_hbm = [p0, p1, p2, p3, p4, p5, p6, p7]
        sid = lax.axis_index("s")
        wid = sid * NC + lax.axis_index("c")
        pltpu.sync_copy(wbs_hbm, wsm)
        iota = lax.iota(jnp.int32, 16)
        jsel = [jnp.full((16,), j, jnp.int32) for j in range(BC)]

        def build_blk(blk, _):
            base = sid * NPW + blk * NB
            for j in range(BC):
                pltpu.async_copy(
                    planes_hbm[j].at[pl.ds(base, NB)], pin[j], bsem)
            for j in range(BC):
                pltpu.make_async_copy(
                    planes_hbm[j].at[pl.ds(0, NB)], pin[j], bsem).wait()

            def ilv(s16, _):
                pts = s16 * 16 + iota
                for j in range(BC):
                    v = pin[j][pl.ds(s16 * 16, 16)]
                    plsc.store_scatter(tblk, [pts, jsel[j]], v)
                return 0

            lax.fori_loop(0, NB // 16, ilv, 0)
            pltpu.sync_copy(tblk, stab.at[pl.ds(base, NB), :])
            return 0

        lax.fori_loop(0, NPW // NB, build_blk, 0)
        plsc.subcore_barrier()

        def off_of(i):
            return jnp.minimum(wid * PW + i * SP, N - SP)

        def prefetch_idx(i, p):
            pltpu.async_copy(
                idx_hbm.at[:, pl.ds(off_of(i), SP)], idx_v[p], isem[p])

        def launch(i, p):
            pltpu.make_async_copy(
                idx_hbm.at[:, pl.ds(0, SP)], idx_v[p], isem[p]).wait()
            for k in range(K):
                pltpu.async_copy(
                    stab.at[idx_v[p].at[k]], bufs[p][k], gsem[p])

        def drain_out(p):
            for j in range(BC):
                pltpu.make_async_copy(
                    out_hbm.at[pl.ds(0, SP)], accT[p].at[j], osem[p]).wait()

        def finish(i, p):
            for k in range(K):
                pltpu.make_async_copy(
                    stab.at[idx_v[p].at[k]], bufs[p][k], gsem[p]).wait()

            @pl.when(i >= 2)
            def _():
                drain_out(p)

            for j in range(BC):
                bias_vj = wsm[K, j]
                wvs = [wsm[k, j] for k in range(K)]

                def seg(s8, _):
                    pts = s8 * 16 + iota
                    acc = bias_vj
                    for k in range(K):
                        v = plsc.load_gather(bufs[p][k], [pts, jsel[j]])
                        acc = acc + v * wvs[k]
                    accT[p][j, pl.ds(s8 * 16, 16)] = acc
                    return 0

                lax.fori_loop(0, SP // 16, seg, 0)
            off = off_of(i)
            for j in range(BC):
                pltpu.async_copy(
                    accT[p].at[j], out_hbm.at[pl.ds(j * N + off, SP)], osem[p])

        prefetch_idx(0, 0)
        launch(0, 0)
        prefetch_idx(1, 1)

        def outer(t, _):
            prefetch_idx(2 * t + 2, 0)
            launch(2 * t + 1, 1)
            finish(2 * t, 0)
            launch(2 * t + 2, 0)
            prefetch_idx(2 * t + 3, 1)
            finish(2 * t + 1, 1)
            return 0

        lax.fori_loop(0, NT - 1, outer, 0)
        launch(NSTEP - 1, 1)
        finish(NSTEP - 2, 0)
        finish(NSTEP - 1, 1)
        drain_out(0)
        drain_out(1)

    return body(*planes, idx_raw, wbs)


def kernel(x, seqnumkmin, Wc, bc):
    xp = x.reshape(B, Q, C, H * W)
    planes = [xp[b, :, c, :].reshape(-1) for b in range(B) for c in range(C)]
    w = Wc.reshape(C, K)
    jc = jnp.arange(BC) % C
    wbs = jnp.concatenate([
        w.T[:, jc],
        bc[jc][None],
    ])[:, :, None] * jnp.ones((1, 1, 16), jnp.float32)
    out = _sc_run(planes, seqnumkmin.T, wbs)
    return out.reshape(B, C, NLAT, MAXJ)
